# baseline (speedup 1.0000x reference)
import functools

import jax
import jax.numpy as jnp
from jax import lax
from jax.experimental import pallas as pl
from jax.experimental.pallas import tpu as pltpu
from jax.experimental.pallas import tpu_sc as plsc

NC = 2
NS = 16
NW = NC * NS
EB = 112
RB = 112
DEPTH = 3
ZR = 1000


def _sc_accumulate(N, d, E2, IP, NPAD):
  assert E2 % (NW * EB) == 0
  nb = E2 // (NW * EB)
  assert nb % DEPTH == 0
  assert IP % (NS * RB) == 0
  rb_per_tile = IP // (NS * RB)
  assert N % ZR == 0 and N // ZR <= NS

  mesh = plsc.VectorSubcoreMesh(core_axis_name="c", subcore_axis_name="s",
                                num_cores=NC, num_subcores=NS)

  @functools.partial(
      pl.kernel,
      mesh=mesh,
      out_type=[
          jax.ShapeDtypeStruct((NC, IP, d), jnp.float32),
          jax.ShapeDtypeStruct((NC * IP,), jnp.float32),
          jax.ShapeDtypeStruct((IP, d), jnp.float32),
      ],
      scratch_types=[
          pltpu.VMEM((nb, EB), jnp.int32),
          [pltpu.VMEM((EB,), jnp.int32) for _ in range(DEPTH)],
          pltpu.VMEM((RB,), jnp.int32),
          pltpu.VMEM((EB,), jnp.float32),
          [pltpu.VMEM((EB, d), jnp.float32) for _ in range(DEPTH)],
          pltpu.VMEM((RB,), jnp.float32),
          pltpu.VMEM((ZR,), jnp.float32),
          pltpu.VMEM_SHARED((16, d), jnp.float32),
          pltpu.VMEM_SHARED((16,), jnp.float32),
          [pltpu.SemaphoreType.DMA for _ in range(DEPTH)],
          [pltpu.SemaphoreType.DMA for _ in range(DEPTH)],
          [pltpu.SemaphoreType.DMA for _ in range(DEPTH)],
          [pltpu.SemaphoreType.DMA for _ in range(DEPTH)],
      ],
  )
  def k(x_hbm, srcm_hbm, dstm_hbm, idx_hbm, zr_hbm, zc_hbm,
        g_hbm, c_hbm, xg_hbm,
        srcc, dstbuf, ibuf, ones, erows, cbuf, zcv,
        acc_sh, cnt_sh, gsem, ssem, csem, isem):
    cid = lax.axis_index("c")
    sid = lax.axis_index("s")
    wid = sid * NC + cid

    pltpu.sync_copy(srcm_hbm.at[wid], srcc)
    for kk in range(EB // 16):
      ones[pl.ds(kk * 16, 16)] = jnp.ones((16,), jnp.float32)

    plsc.subcore_barrier()

    for b in range(DEPTH):
      pltpu.async_copy(dstm_hbm.at[wid, b], dstbuf[b], isem[b])
      pltpu.async_copy(x_hbm.at[srcc.at[b]], erows[b], gsem[b])

    def row_round(o, _):
      for b in range(DEPTH):
        it = o * DEPTH + b
        pltpu.make_async_copy(x_hbm.at[srcc.at[it]], erows[b], gsem[b]).wait()
        pltpu.make_async_copy(dstm_hbm.at[wid, it], dstbuf[b], isem[b]).wait()

        @pl.when(o < nb // DEPTH - 1)
        def _():
          pltpu.async_copy(dstm_hbm.at[wid, it + DEPTH], dstbuf[b], isem[b])
          pltpu.async_copy(x_hbm.at[srcc.at[it + DEPTH]], erows[b], gsem[b])
      return 0
    lax.fori_loop(0, nb // DEPTH, row_round, 0)

    plsc.subcore_barrier()

    for j in range(rb_per_tile):
      rbase = pl.multiple_of(sid * (rb_per_tile * RB) + j * RB, 8)
      pltpu.sync_copy(idx_hbm.at[pl.ds(rbase, RB)], ibuf)
      pltpu.sync_copy(x_hbm.at[ibuf], erows[0])
      pltpu.sync_copy(erows[0], g_hbm.at[cid, pl.ds(rbase, RB)])
      cofs = pl.multiple_of(cid * IP + rbase, 8)
      pltpu.sync_copy(cbuf, c_hbm.at[pl.ds(cofs, RB)])

    @pl.when(cid == 0)
    def _():
      for j in range(rb_per_tile):
        rbase = pl.multiple_of(sid * (rb_per_tile * RB) + j * RB, 8)
        pltpu.sync_copy(idx_hbm.at[pl.ds(rbase, RB)], ibuf)
        pltpu.sync_copy(x_hbm.at[ibuf], erows[1])
        pltpu.sync_copy(erows[1], xg_hbm.at[pl.ds(rbase, RB)])

  return k


def _tc_finish(IP, d, RBLK):
  assert IP % RBLK == 0

  def body(g0, g1, xg, c0, c1, w, b, out):
    s = g0[...] + g1[...] + xg[...]
    c = c0[...] + c1[...] + 1.0
    m = s / c
    out[...] = lax.dot_general(
        m, w[...], dimension_numbers=(((1,), (1,)), ((), ())),
        preferred_element_type=jnp.float32) + b[...]

  return pl.pallas_call(
      body,
      grid=(IP // RBLK,),
      in_specs=[
          pl.BlockSpec((RBLK, d), lambda i: (i, 0)),
          pl.BlockSpec((RBLK, d), lambda i: (i, 0)),
          pl.BlockSpec((RBLK, d), lambda i: (i, 0)),
          pl.BlockSpec((RBLK, 1), lambda i: (i, 0)),
          pl.BlockSpec((RBLK, 1), lambda i: (i, 0)),
          pl.BlockSpec((d, d), lambda i: (0, 0)),
          pl.BlockSpec((1, d), lambda i: (0, 0)),
      ],
      out_specs=pl.BlockSpec((RBLK, d), lambda i: (i, 0)),
      out_shape=jax.ShapeDtypeStruct((IP, d), jnp.float32),
  )


def kernel(x, edge_index, idx, W, b):
  N, d = x.shape
  E = edge_index.shape[1]
  Nsel = idx.shape[0]
  IP = ((Nsel + NS * RB - 1) // (NS * RB)) * (NS * RB)
  EQ = NW * EB * DEPTH
  E2 = ((E + EQ - 1) // EQ) * EQ
  NPAD = N + 8

  src = edge_index[0].astype(jnp.int32)
  dst = edge_index[1].astype(jnp.int32)
  nb = E2 // (NW * EB)
  srcm = jnp.pad(src, (0, E2 - E)).reshape(NW, nb, EB)
  dstm = jnp.pad(dst, (0, E2 - E), constant_values=N).reshape(NW, nb, EB)
  idx32 = idx.astype(jnp.int32)
  idxp = jnp.pad(idx32, (0, IP - Nsel))
  zrows = jnp.zeros((ZR, d), jnp.float32)
  zcnt = jnp.zeros((ZR,), jnp.float32)

  g, c, xg = _sc_accumulate(N, d, E2, IP, NPAD)(
      x, srcm, dstm, idxp, zrows, zcnt)
  out_full = _tc_finish(IP, d, 448)(
      g[0], g[1], xg, c[:IP].reshape(IP, 1), c[IP:].reshape(IP, 1),
      W, b.reshape(1, d))
  return (out_full[:Nsel], idx)


# speedup vs baseline: 1.0707x; 1.0707x over previous
import functools

import jax
import jax.numpy as jnp
from jax import lax
from jax.experimental import pallas as pl
from jax.experimental.pallas import tpu as pltpu
from jax.experimental.pallas import tpu_sc as plsc

NC = 2
NS = 16
NW = NC * NS
FB = 112
EBR = 80
RB = 80
DEPTH = 2
ZR = 1000
MR = 4
PK = 16384
NPADQ = 13 * 16


def _sc_accumulate(N, d, E2, IP, NPAD):
  assert N < PK
  assert E2 % (NW * FB) == 0
  nb = E2 // (NW * FB)
  epw = nb * FB
  assert IP % (NS * RB) == 0
  rb_per_tile = IP // (NS * RB)
  assert N % ZR == 0 and N // ZR <= NS - 1
  PCAP = epw + 16
  QCAP = epw + NPADQ + 16
  ZW = max(MR * FB + 16, ZR)

  mesh = plsc.VectorSubcoreMesh(core_axis_name="c", subcore_axis_name="s",
                                num_cores=NC, num_subcores=NS)

  @functools.partial(
      pl.kernel,
      mesh=mesh,
      out_type=[
          jax.ShapeDtypeStruct((NC, IP, d), jnp.float32),
          jax.ShapeDtypeStruct((NC * IP,), jnp.float32),
          jax.ShapeDtypeStruct((IP, d), jnp.float32),
      ],
      scratch_types=[
          pltpu.VMEM((PCAP,), jnp.int32),
          pltpu.VMEM((QCAP,), jnp.int32),
          [pltpu.VMEM((FB,), jnp.int32) for _ in range(MR)],
          [pltpu.VMEM((EBR,), jnp.int32) for _ in range(DEPTH)],
          [pltpu.VMEM((EBR,), jnp.int32) for _ in range(DEPTH)],
          pltpu.VMEM((RB,), jnp.int32),
          pltpu.VMEM((EBR,), jnp.float32),
          [pltpu.VMEM((EBR, d), jnp.float32) for _ in range(DEPTH)],
          pltpu.VMEM((RB,), jnp.float32),
          pltpu.VMEM((ZW,), jnp.float32),
          pltpu.VMEM_SHARED((NPAD, d), jnp.float32),
          pltpu.VMEM_SHARED((NPAD,), jnp.float32),
          pltpu.VMEM_SHARED((NPAD,), jnp.float32),
          [pltpu.SemaphoreType.DMA for _ in range(DEPTH)],
          [pltpu.SemaphoreType.DMA for _ in range(DEPTH)],
          [pltpu.SemaphoreType.DMA for _ in range(DEPTH)],
          [pltpu.SemaphoreType.DMA for _ in range(MR)],
      ],
  )
  def k(x_hbm, pkm_hbm, idx_hbm, zr_hbm, zc_hbm,
        g_hbm, c_hbm, xg_hbm,
        pkq, flq, dring, srcbuf, dstbuf, ibuf, ones, erows, cbuf, zcv,
        acc_sh, cnt_sh, mrk_sh, gsem, ssem, csem, msem):
    cid = lax.axis_index("c")
    sid = lax.axis_index("s")
    wid = sid * NC + cid
    i32 = jnp.int32

    pltpu.sync_copy(pkm_hbm.at[wid], pkq)
    for kk in range(EBR // 16):
      ones[pl.ds(kk * 16, 16)] = jnp.ones((16,), jnp.float32)
    pltpu.sync_copy(zc_hbm, zcv.at[pl.ds(0, ZR)])

    @pl.when(sid < N // ZR)
    def _():
      zr = pl.multiple_of(sid * ZR, 8)
      pltpu.sync_copy(zr_hbm, acc_sh.at[pl.ds(zr, ZR), :])
      pltpu.sync_copy(zcv.at[pl.ds(0, ZR)], cnt_sh.at[pl.ds(zr, ZR)])
      pltpu.sync_copy(zcv.at[pl.ds(0, ZR)], mrk_sh.at[pl.ds(zr, ZR)])

    @pl.when(sid == NS - 1)
    def _():
      pltpu.sync_copy(zcv.at[pl.ds(0, NPAD - N)],
                      cnt_sh.at[pl.ds(N, NPAD - N)])
      pltpu.sync_copy(zcv.at[pl.ds(0, NPAD - N)],
                      mrk_sh.at[pl.ds(N, NPAD - N)])
    plsc.subcore_barrier()

    for j in range(rb_per_tile):
      rbase = pl.multiple_of(sid * (rb_per_tile * RB) + j * RB, 8)
      pltpu.sync_copy(idx_hbm.at[pl.ds(rbase, RB)], ibuf)
      pltpu.sync_copy(ones, mrk_sh.at[ibuf])
    plsc.subcore_barrier()

    def unpack_dst(slot, j):
      for kk in range(FB // 16):
        pk16 = pkq[pl.ds(j * FB + kk * 16, 16)]
        dring[slot][pl.ds(kk * 16, 16)] = lax.rem(pk16, PK)

    def fire_mask(slot):
      pltpu.async_copy(mrk_sh.at[dring[slot]],
                       zcv.at[pl.ds(slot * FB, FB)], msem[slot])

    for j in range(MR):
      unpack_dst(j, jnp.int32(j))
      fire_mask(j)

    def filt_step(j, off):
      for slot in range(MR):
        @pl.when(lax.rem(j, MR) == slot)
        def _():
          pltpu.make_async_copy(mrk_sh.at[dring[slot]],
                                zcv.at[pl.ds(slot * FB, FB)],
                                msem[slot]).wait()

      def edge(e, off2):
        sl = lax.rem(j, MR)
        mval = zcv[pl.ds(sl * FB + e, 16)][0]
        pval = pkq[pl.ds(j * FB + e, 16)][0]
        flq[pl.ds(off2, 16)] = jnp.full((16,), pval, i32)
        return off2 + jnp.where(mval > 0.5, 1, 0)
      off = lax.fori_loop(0, FB, edge, off)

      for slot in range(MR):
        @pl.when((lax.rem(j, MR) == slot) & (j + MR < nb))
        def _():
          unpack_dst(slot, j + MR)
          fire_mask(slot)
      return off
    off = lax.fori_loop(0, nb, filt_step, jnp.int32(0))

    for t in range(NPADQ // 16):
      flq[pl.ds(off + t * 16, 16)] = jnp.full((16,), N, i32)
    nbk = jnp.maximum((off + EBR - 1) // EBR, DEPTH)

    def unpack_row(b, it):
      for kk in range(EBR // 16):
        pk16 = flq[pl.ds(it * EBR + kk * 16, 16)]
        srcbuf[b][pl.ds(kk * 16, 16)] = lax.div(pk16, PK)
        dstbuf[b][pl.ds(kk * 16, 16)] = lax.rem(pk16, PK)

    for b in range(DEPTH):
      unpack_row(b, jnp.int32(b))
      pltpu.async_copy(x_hbm.at[srcbuf[b]], erows[b], gsem[b])

    def row_round(it, _):
      for b in range(DEPTH):
        @pl.when(lax.rem(it, DEPTH) == b)
        def _():
          pltpu.make_async_copy(x_hbm.at[srcbuf[b]], erows[b],
                                gsem[b]).wait()
          pltpu.async_copy(erows[b], acc_sh.at[dstbuf[b]], ssem[b], add=True)
          pltpu.async_copy(ones, cnt_sh.at[dstbuf[b]], csem[b], add=True)

          @pl.when(it + DEPTH < nbk)
          def _():
            pltpu.make_async_copy(erows[b], acc_sh.at[dstbuf[b]],
                                  ssem[b]).wait()
            pltpu.make_async_copy(ones, cnt_sh.at[dstbuf[b]],
                                  csem[b]).wait()
            unpack_row(b, it + DEPTH)
            pltpu.async_copy(x_hbm.at[srcbuf[b]], erows[b], gsem[b])
      return 0
    lax.fori_loop(0, nbk, row_round, 0)

    for b in range(DEPTH):
      pltpu.make_async_copy(erows[b], acc_sh.at[dstbuf[b]], ssem[b]).wait()
      pltpu.make_async_copy(ones, cnt_sh.at[dstbuf[b]], csem[b]).wait()
    plsc.subcore_barrier()

    for j in range(rb_per_tile):
      rbase = pl.multiple_of(sid * (rb_per_tile * RB) + j * RB, 8)
      pltpu.sync_copy(idx_hbm.at[pl.ds(rbase, RB)], ibuf)
      pltpu.sync_copy(acc_sh.at[ibuf], erows[0])
      pltpu.sync_copy(erows[0], g_hbm.at[cid, pl.ds(rbase, RB)])
      pltpu.sync_copy(cnt_sh.at[ibuf], cbuf)
      cofs = pl.multiple_of(cid * IP + rbase, 8)
      pltpu.sync_copy(cbuf, c_hbm.at[pl.ds(cofs, RB)])

    @pl.when(cid == 0)
    def _():
      for j in range(rb_per_tile):
        rbase = pl.multiple_of(sid * (rb_per_tile * RB) + j * RB, 8)
        pltpu.sync_copy(idx_hbm.at[pl.ds(rbase, RB)], ibuf)
        pltpu.sync_copy(x_hbm.at[ibuf], erows[1])
        pltpu.sync_copy(erows[1], xg_hbm.at[pl.ds(rbase, RB)])

  return k


def _tc_finish(IP, d, RBLK):
  assert IP % RBLK == 0

  def body(g0, g1, xg, c0, c1, w, b, out):
    s = g0[...] + g1[...] + xg[...]
    c = c0[...] + c1[...] + 1.0
    m = s / c
    out[...] = lax.dot_general(
        m, w[...], dimension_numbers=(((1,), (1,)), ((), ())),
        preferred_element_type=jnp.float32) + b[...]

  return pl.pallas_call(
      body,
      grid=(IP // RBLK,),
      in_specs=[
          pl.BlockSpec((RBLK, d), lambda i: (i, 0)),
          pl.BlockSpec((RBLK, d), lambda i: (i, 0)),
          pl.BlockSpec((RBLK, d), lambda i: (i, 0)),
          pl.BlockSpec((RBLK, 1), lambda i: (i, 0)),
          pl.BlockSpec((RBLK, 1), lambda i: (i, 0)),
          pl.BlockSpec((d, d), lambda i: (0, 0)),
          pl.BlockSpec((1, d), lambda i: (0, 0)),
      ],
      out_specs=pl.BlockSpec((RBLK, d), lambda i: (i, 0)),
      out_shape=jax.ShapeDtypeStruct((IP, d), jnp.float32),
  )


def kernel(x, edge_index, idx, W, b):
  N, d = x.shape
  E = edge_index.shape[1]
  Nsel = idx.shape[0]
  IP = ((Nsel + NS * RB - 1) // (NS * RB)) * (NS * RB)
  EQ = NW * FB
  E2 = ((E + EQ - 1) // EQ) * EQ
  NPAD = N + 8

  src = edge_index[0].astype(jnp.int32)
  dst = edge_index[1].astype(jnp.int32)
  nb = E2 // (NW * FB)
  packed = src * PK + dst
  pkm = jnp.pad(jnp.pad(packed, (0, E2 - E), constant_values=N)
                .reshape(NW, nb * FB), ((0, 0), (0, 16)),
                constant_values=N)
  idx32 = idx.astype(jnp.int32)
  idxp = jnp.pad(idx32, (0, IP - Nsel))
  zrows = jnp.zeros((ZR, d), jnp.float32)
  zcnt = jnp.zeros((ZR,), jnp.float32)

  g, c, xg = _sc_accumulate(N, d, E2, IP, NPAD)(
      x, pkm, idxp, zrows, zcnt)
  out_full = _tc_finish(IP, d, 640)(
      g[0], g[1], xg, c[:IP].reshape(IP, 1), c[IP:].reshape(IP, 1),
      W, b.reshape(1, d))
  return (out_full[:Nsel], idx)


# speedup vs baseline: 1.1774x; 1.0996x over previous
import functools

import jax
import jax.numpy as jnp
from jax import lax
from jax.experimental import pallas as pl
from jax.experimental.pallas import tpu as pltpu
from jax.experimental.pallas import tpu_sc as plsc

NC = 2
NS = 16
NW = NC * NS
FB = 112
EBR = 80
RB = 80
DEPTH = 2
ZR = 1000
MR = 4
PK = 16384
NPADQ = 13 * 16


def _sc_accumulate(N, d, E2, IP, NPAD):
  assert N < PK
  assert E2 % (NW * FB) == 0
  nb = E2 // (NW * FB)
  epw = nb * FB
  assert IP % (NS * RB) == 0
  rb_per_tile = IP // (NS * RB)
  assert N % ZR == 0 and N // ZR <= NS - 1
  PCAP = epw + 16
  QCAP = epw + NPADQ + 16
  ZW = max(MR * FB + 16, ZR)

  mesh = plsc.VectorSubcoreMesh(core_axis_name="c", subcore_axis_name="s",
                                num_cores=NC, num_subcores=NS)

  @functools.partial(
      pl.kernel,
      mesh=mesh,
      out_type=[
          jax.ShapeDtypeStruct((NC, IP, d), jnp.float32),
          jax.ShapeDtypeStruct((NC * IP,), jnp.float32),
          jax.ShapeDtypeStruct((IP, d), jnp.float32),
      ],
      scratch_types=[
          pltpu.VMEM((PCAP,), jnp.int32),
          pltpu.VMEM((QCAP,), jnp.int32),
          [pltpu.VMEM((FB,), jnp.int32) for _ in range(MR)],
          [pltpu.VMEM((EBR,), jnp.int32) for _ in range(DEPTH)],
          [pltpu.VMEM((EBR,), jnp.int32) for _ in range(DEPTH)],
          pltpu.VMEM((RB,), jnp.int32),
          pltpu.VMEM((EBR,), jnp.float32),
          [pltpu.VMEM((EBR, d), jnp.float32) for _ in range(DEPTH)],
          pltpu.VMEM((RB,), jnp.float32),
          pltpu.VMEM((ZW,), jnp.float32),
          pltpu.VMEM_SHARED((NPAD, d), jnp.float32),
          pltpu.VMEM_SHARED((NPAD,), jnp.float32),
          pltpu.VMEM_SHARED((NPAD,), jnp.float32),
          [pltpu.SemaphoreType.DMA for _ in range(DEPTH)],
          [pltpu.SemaphoreType.DMA for _ in range(DEPTH)],
          [pltpu.SemaphoreType.DMA for _ in range(DEPTH)],
          [pltpu.SemaphoreType.DMA for _ in range(MR)],
      ],
  )
  def k(x_hbm, pkm_hbm, idx_hbm, zr_hbm, zc_hbm,
        g_hbm, c_hbm, xg_hbm,
        pkq, flq, dring, srcbuf, dstbuf, ibuf, ones, erows, cbuf, zcv,
        acc_sh, cnt_sh, mrk_sh, gsem, ssem, csem, msem):
    cid = lax.axis_index("c")
    sid = lax.axis_index("s")
    wid = sid * NC + cid
    i32 = jnp.int32

    pltpu.sync_copy(pkm_hbm.at[wid], pkq)
    for kk in range(EBR // 16):
      ones[pl.ds(kk * 16, 16)] = jnp.ones((16,), jnp.float32)
    pltpu.sync_copy(zc_hbm, zcv.at[pl.ds(0, ZR)])

    @pl.when(sid < N // ZR)
    def _():
      zr = pl.multiple_of(sid * ZR, 8)
      pltpu.sync_copy(zr_hbm, acc_sh.at[pl.ds(zr, ZR), :])
      pltpu.sync_copy(zcv.at[pl.ds(0, ZR)], cnt_sh.at[pl.ds(zr, ZR)])
      pltpu.sync_copy(zcv.at[pl.ds(0, ZR)], mrk_sh.at[pl.ds(zr, ZR)])

    @pl.when(sid == NS - 1)
    def _():
      pltpu.sync_copy(zcv.at[pl.ds(0, NPAD - N)],
                      cnt_sh.at[pl.ds(N, NPAD - N)])
      pltpu.sync_copy(zcv.at[pl.ds(0, NPAD - N)],
                      mrk_sh.at[pl.ds(N, NPAD - N)])
    plsc.subcore_barrier()

    for j in range(rb_per_tile):
      rbase = pl.multiple_of(sid * (rb_per_tile * RB) + j * RB, 8)
      pltpu.sync_copy(idx_hbm.at[pl.ds(rbase, RB)], ibuf)
      pltpu.sync_copy(ones, mrk_sh.at[ibuf])
    plsc.subcore_barrier()

    def unpack_dst(slot, j):
      for kk in range(FB // 16):
        pk16 = pkq[pl.ds(j * FB + kk * 16, 16)]
        dring[slot][pl.ds(kk * 16, 16)] = lax.rem(pk16, PK)

    def fire_mask(slot):
      pltpu.async_copy(mrk_sh.at[dring[slot]],
                       zcv.at[pl.ds(slot * FB, FB)], msem[slot])

    for j in range(MR):
      unpack_dst(j, jnp.int32(j))
      fire_mask(j)

    def filt_step(j, off):
      for slot in range(MR):
        @pl.when(lax.rem(j, MR) == slot)
        def _():
          pltpu.make_async_copy(mrk_sh.at[dring[slot]],
                                zcv.at[pl.ds(slot * FB, FB)],
                                msem[slot]).wait()

      sl = lax.rem(j, MR)
      jbase = j * FB
      sbase = sl * FB
      for e in range(FB):
        mval = zcv[pl.ds(sbase + e, 16)][0]
        pval = pkq[pl.ds(jbase + e, 16)][0]
        flq[pl.ds(off, 16)] = jnp.full((16,), pval, i32)
        off = off + jnp.where(mval > 0.5, 1, 0)

      for slot in range(MR):
        @pl.when((lax.rem(j, MR) == slot) & (j + MR < nb))
        def _():
          unpack_dst(slot, j + MR)
          fire_mask(slot)
      return off
    off = lax.fori_loop(0, nb, filt_step, jnp.int32(0))

    for t in range(NPADQ // 16):
      flq[pl.ds(off + t * 16, 16)] = jnp.full((16,), N, i32)
    nbk = jnp.maximum((off + EBR - 1) // EBR, DEPTH)

    def unpack_row(b, it):
      for kk in range(EBR // 16):
        pk16 = flq[pl.ds(it * EBR + kk * 16, 16)]
        srcbuf[b][pl.ds(kk * 16, 16)] = lax.div(pk16, PK)
        dstbuf[b][pl.ds(kk * 16, 16)] = lax.rem(pk16, PK)

    for b in range(DEPTH):
      unpack_row(b, jnp.int32(b))
      pltpu.async_copy(x_hbm.at[srcbuf[b]], erows[b], gsem[b])

    def row_round(it, _):
      for b in range(DEPTH):
        @pl.when(lax.rem(it, DEPTH) == b)
        def _():
          pltpu.make_async_copy(x_hbm.at[srcbuf[b]], erows[b],
                                gsem[b]).wait()
          pltpu.async_copy(erows[b], acc_sh.at[dstbuf[b]], ssem[b], add=True)
          pltpu.async_copy(ones, cnt_sh.at[dstbuf[b]], csem[b], add=True)

          @pl.when(it + DEPTH < nbk)
          def _():
            pltpu.make_async_copy(erows[b], acc_sh.at[dstbuf[b]],
                                  ssem[b]).wait()
            pltpu.make_async_copy(ones, cnt_sh.at[dstbuf[b]],
                                  csem[b]).wait()
            unpack_row(b, it + DEPTH)
            pltpu.async_copy(x_hbm.at[srcbuf[b]], erows[b], gsem[b])
      return 0
    lax.fori_loop(0, nbk, row_round, 0)

    for b in range(DEPTH):
      pltpu.make_async_copy(erows[b], acc_sh.at[dstbuf[b]], ssem[b]).wait()
      pltpu.make_async_copy(ones, cnt_sh.at[dstbuf[b]], csem[b]).wait()
    plsc.subcore_barrier()

    for j in range(rb_per_tile):
      rbase = pl.multiple_of(sid * (rb_per_tile * RB) + j * RB, 8)
      pltpu.sync_copy(idx_hbm.at[pl.ds(rbase, RB)], ibuf)
      pltpu.sync_copy(acc_sh.at[ibuf], erows[0])
      pltpu.sync_copy(erows[0], g_hbm.at[cid, pl.ds(rbase, RB)])
      pltpu.sync_copy(cnt_sh.at[ibuf], cbuf)
      cofs = pl.multiple_of(cid * IP + rbase, 8)
      pltpu.sync_copy(cbuf, c_hbm.at[pl.ds(cofs, RB)])

    @pl.when(cid == 0)
    def _():
      for j in range(rb_per_tile):
        rbase = pl.multiple_of(sid * (rb_per_tile * RB) + j * RB, 8)
        pltpu.sync_copy(idx_hbm.at[pl.ds(rbase, RB)], ibuf)
        pltpu.sync_copy(x_hbm.at[ibuf], erows[1])
        pltpu.sync_copy(erows[1], xg_hbm.at[pl.ds(rbase, RB)])

  return k


def _tc_finish(IP, d, RBLK):
  assert IP % RBLK == 0

  def body(g0, g1, xg, c0, c1, w, b, out):
    s = g0[...] + g1[...] + xg[...]
    c = c0[...] + c1[...] + 1.0
    m = s / c
    out[...] = lax.dot_general(
        m, w[...], dimension_numbers=(((1,), (1,)), ((), ())),
        preferred_element_type=jnp.float32) + b[...]

  return pl.pallas_call(
      body,
      grid=(IP // RBLK,),
      in_specs=[
          pl.BlockSpec((RBLK, d), lambda i: (i, 0)),
          pl.BlockSpec((RBLK, d), lambda i: (i, 0)),
          pl.BlockSpec((RBLK, d), lambda i: (i, 0)),
          pl.BlockSpec((RBLK, 1), lambda i: (i, 0)),
          pl.BlockSpec((RBLK, 1), lambda i: (i, 0)),
          pl.BlockSpec((d, d), lambda i: (0, 0)),
          pl.BlockSpec((1, d), lambda i: (0, 0)),
      ],
      out_specs=pl.BlockSpec((RBLK, d), lambda i: (i, 0)),
      out_shape=jax.ShapeDtypeStruct((IP, d), jnp.float32),
  )


def kernel(x, edge_index, idx, W, b):
  N, d = x.shape
  E = edge_index.shape[1]
  Nsel = idx.shape[0]
  IP = ((Nsel + NS * RB - 1) // (NS * RB)) * (NS * RB)
  EQ = NW * FB
  E2 = ((E + EQ - 1) // EQ) * EQ
  NPAD = N + 8

  src = edge_index[0].astype(jnp.int32)
  dst = edge_index[1].astype(jnp.int32)
  nb = E2 // (NW * FB)
  packed = src * PK + dst
  pkm = jnp.pad(jnp.pad(packed, (0, E2 - E), constant_values=N)
                .reshape(NW, nb * FB), ((0, 0), (0, 16)),
                constant_values=N)
  idx32 = idx.astype(jnp.int32)
  idxp = jnp.pad(idx32, (0, IP - Nsel))
  zrows = jnp.zeros((ZR, d), jnp.float32)
  zcnt = jnp.zeros((ZR,), jnp.float32)

  g, c, xg = _sc_accumulate(N, d, E2, IP, NPAD)(
      x, pkm, idxp, zrows, zcnt)
  out_full = _tc_finish(IP, d, 640)(
      g[0], g[1], xg, c[:IP].reshape(IP, 1), c[IP:].reshape(IP, 1),
      W, b.reshape(1, d))
  return (out_full[:Nsel], idx)


# speedup vs baseline: 1.2128x; 1.0301x over previous
import functools

import jax
import jax.numpy as jnp
from jax import lax
from jax.experimental import pallas as pl
from jax.experimental.pallas import tpu as pltpu
from jax.experimental.pallas import tpu_sc as plsc

NC = 2
NS = 16
NW = NC * NS
FB = 112
EBR = 80
RB = 80
DEPTH = 2
ZR = 1000
MR = 4
PK = 16384
NPADQ = 13 * 16


def _sc_accumulate(N, d, E2, IP, NPAD):
  assert N < PK
  assert E2 % (NW * FB) == 0
  nb = E2 // (NW * FB)
  epw = nb * FB
  assert IP % (NS * RB) == 0
  rb_per_tile = IP // (NS * RB)
  assert N % ZR == 0 and N // ZR <= NS - 1
  PCAP = epw + 16
  QCAP = epw + NPADQ + 16
  ZW = max(MR * FB + 16, ZR)

  mesh = plsc.VectorSubcoreMesh(core_axis_name="c", subcore_axis_name="s",
                                num_cores=NC, num_subcores=NS)

  @functools.partial(
      pl.kernel,
      mesh=mesh,
      out_type=[
          jax.ShapeDtypeStruct((NC, IP, d), jnp.float32),
          jax.ShapeDtypeStruct((NC * IP,), jnp.float32),
          jax.ShapeDtypeStruct((IP, d), jnp.float32),
      ],
      scratch_types=[
          pltpu.VMEM((PCAP,), jnp.int32),
          pltpu.VMEM((QCAP,), jnp.int32),
          [pltpu.VMEM((FB,), jnp.int32) for _ in range(MR)],
          pltpu.VMEM((MR * FB + 16,), jnp.int32),
          [pltpu.VMEM((EBR,), jnp.int32) for _ in range(DEPTH)],
          [pltpu.VMEM((EBR,), jnp.int32) for _ in range(DEPTH)],
          pltpu.VMEM((RB,), jnp.int32),
          pltpu.VMEM((EBR,), jnp.float32),
          [pltpu.VMEM((EBR, d), jnp.float32) for _ in range(DEPTH)],
          pltpu.VMEM((RB,), jnp.float32),
          pltpu.VMEM((ZW,), jnp.float32),
          pltpu.VMEM_SHARED((NPAD, d), jnp.float32),
          pltpu.VMEM_SHARED((NPAD,), jnp.float32),
          pltpu.VMEM_SHARED((NPAD,), jnp.float32),
          [pltpu.SemaphoreType.DMA for _ in range(DEPTH)],
          [pltpu.SemaphoreType.DMA for _ in range(DEPTH)],
          [pltpu.SemaphoreType.DMA for _ in range(DEPTH)],
          [pltpu.SemaphoreType.DMA for _ in range(MR)],
      ],
  )
  def k(x_hbm, pkm_hbm, idx_hbm, zr_hbm, zc_hbm,
        g_hbm, c_hbm, xg_hbm,
        pkq, flq, dring, mring, srcbuf, dstbuf, ibuf, ones, erows, cbuf, zcv,
        acc_sh, cnt_sh, mrk_sh, gsem, ssem, csem, msem):
    cid = lax.axis_index("c")
    sid = lax.axis_index("s")
    wid = sid * NC + cid
    i32 = jnp.int32

    pltpu.sync_copy(pkm_hbm.at[wid], pkq)
    for kk in range(EBR // 16):
      ones[pl.ds(kk * 16, 16)] = jnp.ones((16,), jnp.float32)
    pltpu.sync_copy(zc_hbm, zcv.at[pl.ds(0, ZR)])

    @pl.when(sid < N // ZR)
    def _():
      zr = pl.multiple_of(sid * ZR, 8)
      pltpu.sync_copy(zr_hbm, acc_sh.at[pl.ds(zr, ZR), :])
      pltpu.sync_copy(zcv.at[pl.ds(0, ZR)], cnt_sh.at[pl.ds(zr, ZR)])
      pltpu.sync_copy(zcv.at[pl.ds(0, ZR)], mrk_sh.at[pl.ds(zr, ZR)])

    @pl.when(sid == NS - 1)
    def _():
      pltpu.sync_copy(zcv.at[pl.ds(0, NPAD - N)],
                      cnt_sh.at[pl.ds(N, NPAD - N)])
      pltpu.sync_copy(zcv.at[pl.ds(0, NPAD - N)],
                      mrk_sh.at[pl.ds(N, NPAD - N)])
    plsc.subcore_barrier()

    for j in range(rb_per_tile):
      rbase = pl.multiple_of(sid * (rb_per_tile * RB) + j * RB, 8)
      pltpu.sync_copy(idx_hbm.at[pl.ds(rbase, RB)], ibuf)
      pltpu.sync_copy(ones, mrk_sh.at[ibuf])
    plsc.subcore_barrier()

    def unpack_dst(slot, j):
      for kk in range(FB // 16):
        pk16 = pkq[pl.ds(j * FB + kk * 16, 16)]
        dring[slot][pl.ds(kk * 16, 16)] = lax.rem(pk16, PK)

    def fire_mask(slot):
      pltpu.async_copy(mrk_sh.at[dring[slot]],
                       zcv.at[pl.ds(slot * FB, FB)], msem[slot])

    for j in range(MR):
      unpack_dst(j, jnp.int32(j))
      fire_mask(j)

    def filt_step(j, off):
      for slot in range(MR):
        @pl.when(lax.rem(j, MR) == slot)
        def _():
          pltpu.make_async_copy(mrk_sh.at[dring[slot]],
                                zcv.at[pl.ds(slot * FB, FB)],
                                msem[slot]).wait()

      sl = lax.rem(j, MR)
      sbase = sl * FB
      for kk in range(FB // 16):
        m16 = zcv[pl.ds(sbase + kk * 16, 16)]
        mring[pl.ds(sbase + kk * 16, 16)] = m16.astype(i32)

      jbase = j * FB
      for e in range(FB):
        mval = mring[pl.ds(sbase + e, 16)][0]
        pval = pkq[pl.ds(jbase + e, 16)][0]
        flq[pl.ds(off, 16)] = jnp.full((16,), pval, i32)
        off = off + mval

      for slot in range(MR):
        @pl.when((lax.rem(j, MR) == slot) & (j + MR < nb))
        def _():
          unpack_dst(slot, j + MR)
          fire_mask(slot)
      return off
    off = lax.fori_loop(0, nb, filt_step, jnp.int32(0))

    for t in range(NPADQ // 16):
      flq[pl.ds(off + t * 16, 16)] = jnp.full((16,), N, i32)
    nbk = jnp.maximum((off + EBR - 1) // EBR, DEPTH)

    def unpack_row(b, it):
      for kk in range(EBR // 16):
        pk16 = flq[pl.ds(it * EBR + kk * 16, 16)]
        srcbuf[b][pl.ds(kk * 16, 16)] = lax.div(pk16, PK)
        dstbuf[b][pl.ds(kk * 16, 16)] = lax.rem(pk16, PK)

    for b in range(DEPTH):
      unpack_row(b, jnp.int32(b))
      pltpu.async_copy(x_hbm.at[srcbuf[b]], erows[b], gsem[b])

    def row_round(it, _):
      for b in range(DEPTH):
        @pl.when(lax.rem(it, DEPTH) == b)
        def _():
          pltpu.make_async_copy(x_hbm.at[srcbuf[b]], erows[b],
                                gsem[b]).wait()
          pltpu.async_copy(erows[b], acc_sh.at[dstbuf[b]], ssem[b], add=True)
          pltpu.async_copy(ones, cnt_sh.at[dstbuf[b]], csem[b], add=True)

          @pl.when(it + DEPTH < nbk)
          def _():
            pltpu.make_async_copy(erows[b], acc_sh.at[dstbuf[b]],
                                  ssem[b]).wait()
            pltpu.make_async_copy(ones, cnt_sh.at[dstbuf[b]],
                                  csem[b]).wait()
            unpack_row(b, it + DEPTH)
            pltpu.async_copy(x_hbm.at[srcbuf[b]], erows[b], gsem[b])
      return 0
    lax.fori_loop(0, nbk, row_round, 0)

    for b in range(DEPTH):
      pltpu.make_async_copy(erows[b], acc_sh.at[dstbuf[b]], ssem[b]).wait()
      pltpu.make_async_copy(ones, cnt_sh.at[dstbuf[b]], csem[b]).wait()
    plsc.subcore_barrier()

    for j in range(rb_per_tile):
      rbase = pl.multiple_of(sid * (rb_per_tile * RB) + j * RB, 8)
      pltpu.sync_copy(idx_hbm.at[pl.ds(rbase, RB)], ibuf)
      pltpu.sync_copy(acc_sh.at[ibuf], erows[0])
      pltpu.sync_copy(erows[0], g_hbm.at[cid, pl.ds(rbase, RB)])
      pltpu.sync_copy(cnt_sh.at[ibuf], cbuf)
      cofs = pl.multiple_of(cid * IP + rbase, 8)
      pltpu.sync_copy(cbuf, c_hbm.at[pl.ds(cofs, RB)])

    @pl.when(cid == 0)
    def _():
      for j in range(rb_per_tile):
        rbase = pl.multiple_of(sid * (rb_per_tile * RB) + j * RB, 8)
        pltpu.sync_copy(idx_hbm.at[pl.ds(rbase, RB)], ibuf)
        pltpu.sync_copy(x_hbm.at[ibuf], erows[1])
        pltpu.sync_copy(erows[1], xg_hbm.at[pl.ds(rbase, RB)])

  return k


def _tc_finish(IP, d, RBLK):
  assert IP % RBLK == 0

  def body(g0, g1, xg, c0, c1, w, b, out):
    s = g0[...] + g1[...] + xg[...]
    c = c0[...] + c1[...] + 1.0
    m = s / c
    out[...] = lax.dot_general(
        m, w[...], dimension_numbers=(((1,), (1,)), ((), ())),
        preferred_element_type=jnp.float32) + b[...]

  return pl.pallas_call(
      body,
      grid=(IP // RBLK,),
      in_specs=[
          pl.BlockSpec((RBLK, d), lambda i: (i, 0)),
          pl.BlockSpec((RBLK, d), lambda i: (i, 0)),
          pl.BlockSpec((RBLK, d), lambda i: (i, 0)),
          pl.BlockSpec((RBLK, 1), lambda i: (i, 0)),
          pl.BlockSpec((RBLK, 1), lambda i: (i, 0)),
          pl.BlockSpec((d, d), lambda i: (0, 0)),
          pl.BlockSpec((1, d), lambda i: (0, 0)),
      ],
      out_specs=pl.BlockSpec((RBLK, d), lambda i: (i, 0)),
      out_shape=jax.ShapeDtypeStruct((IP, d), jnp.float32),
  )


def kernel(x, edge_index, idx, W, b):
  N, d = x.shape
  E = edge_index.shape[1]
  Nsel = idx.shape[0]
  IP = ((Nsel + NS * RB - 1) // (NS * RB)) * (NS * RB)
  EQ = NW * FB
  E2 = ((E + EQ - 1) // EQ) * EQ
  NPAD = N + 8

  src = edge_index[0].astype(jnp.int32)
  dst = edge_index[1].astype(jnp.int32)
  nb = E2 // (NW * FB)
  packed = src * PK + dst
  pkm = jnp.pad(jnp.pad(packed, (0, E2 - E), constant_values=N)
                .reshape(NW, nb * FB), ((0, 0), (0, 16)),
                constant_values=N)
  idx32 = idx.astype(jnp.int32)
  idxp = jnp.pad(idx32, (0, IP - Nsel))
  zrows = jnp.zeros((ZR, d), jnp.float32)
  zcnt = jnp.zeros((ZR,), jnp.float32)

  g, c, xg = _sc_accumulate(N, d, E2, IP, NPAD)(
      x, pkm, idxp, zrows, zcnt)
  out_full = _tc_finish(IP, d, 640)(
      g[0], g[1], xg, c[:IP].reshape(IP, 1), c[IP:].reshape(IP, 1),
      W, b.reshape(1, d))
  return (out_full[:Nsel], idx)
